# Initial kernel scaffold; baseline (speedup 1.0000x reference)
import jax
import jax.numpy as jnp
from jax.experimental import pallas as pl


def kernel(x, src_x, dst_x, edge_index, Wq, bq, Wk, bk, Wv, bv, Wo, bo, W1, b1, W2, b2, ln1_g, ln1_b, ln2_g, ln2_b):
    raise NotImplementedError("write your pallas kernel here")


# speedup vs baseline: 13.7159x; 13.7159x over previous
import jax
import jax.numpy as jnp
from jax import lax
from jax.experimental import pallas as pl
from jax.experimental.pallas import tpu as pltpu
from jax.experimental.pallas import tpu_sc as plsc

N = 10000
E = 320000
EDIM = 128
NDIM = 128
H = 8
DK = 16
NC = 2
NS = 16
NW = NC * NS
EPW = E // NW
C = 40
NCHUNK = EPW // C
ROW_W = H + 1


_BR = 1000


def _qkv_body(x_ref, w_ref, b_ref, kv_ref, q_ref):
    y = jnp.dot(x_ref[...], w_ref[...], preferred_element_type=jnp.float32)
    y = y + b_ref[...]
    kv_ref[...] = y[:, : 2 * NDIM]
    q_ref[...] = y[:, 2 * NDIM :]


_qkv_call = pl.pallas_call(
    _qkv_body,
    grid=(N // _BR,),
    in_specs=[
        pl.BlockSpec((_BR, EDIM), lambda i: (i, 0)),
        pl.BlockSpec((EDIM, 3 * NDIM), lambda i: (0, 0)),
        pl.BlockSpec((1, 3 * NDIM), lambda i: (0, 0)),
    ],
    out_specs=[
        pl.BlockSpec((_BR, 2 * NDIM), lambda i: (i, 0)),
        pl.BlockSpec((_BR, NDIM), lambda i: (i, 0)),
    ],
    out_shape=[
        jax.ShapeDtypeStruct((N, 2 * NDIM), jnp.float32),
        jax.ShapeDtypeStruct((N, NDIM), jnp.float32),
    ],
)


SZ = 320
OUTR = NW * SZ
MAXE = 11264
SCCH = 1280
CE = 128
ZPAD = SZ * H


def _edge_body(kv_hbm, q_hbm, src_hbm, dst_hbm, zerof_hbm, zeroi_hbm,
               wv_hbm, z_hbm,
               src_l, dst_l, sbuf, dbuf, kv_buf, q_buf, acc, z_flat,
               sem_kv, sem_q):
    c = lax.axis_index("c")
    s = lax.axis_index("s")
    wid = c * NS + s
    start = wid * SZ

    pltpu.sync_copy(zerof_hbm, acc)
    pltpu.sync_copy(zeroi_hbm, src_l)
    pltpu.sync_copy(zeroi_hbm, dst_l)

    lane = lax.iota(jnp.int32, DK)
    shuf = [jnp.bitwise_and(lane + sh, DK - 1) for sh in (8, 4, 2, 1)]
    zerov = jnp.zeros((DK,), jnp.float32)

    def zloop(t, carry):
        z_flat[pl.ds(t * DK, DK)] = zerov
        return carry

    lax.fori_loop(0, ZPAD // DK, zloop, 0)

    def scan_blk(b, off):
        pltpu.sync_copy(src_hbm.at[pl.ds(b * SCCH, SCCH)], sbuf)
        pltpu.sync_copy(dst_hbm.at[pl.ds(b * SCCH, SCCH)], dbuf)

        def scan16(t, off2):
            dv = dbuf[pl.ds(t * DK, DK)]
            sv = sbuf[pl.ds(t * DK, DK)]
            loc = dv - start
            m = jnp.logical_and(loc >= 0, loc < SZ)
            pos = off2 + plsc.cumsum(m.astype(jnp.int32)) - 1
            plsc.store_scatter(dst_l, [pos], dv, mask=m)
            plsc.store_scatter(src_l, [pos], sv, mask=m)
            return off2 + plsc.all_reduce_population_count(m)

        return lax.fori_loop(0, SCCH // DK, scan16, off)

    off = lax.fori_loop(0, E // SCCH, scan_blk, jnp.zeros((DK,), jnp.int32))
    nloc = off[0]

    def chunk_body(i, carry):
        base = i * CE
        cp_kv = pltpu.async_copy(kv_hbm.at[src_l.at[pl.ds(base, CE)]],
                                 kv_buf, sem_kv)
        cp_q = pltpu.async_copy(q_hbm.at[dst_l.at[pl.ds(base, CE)]],
                                q_buf, sem_q)
        cp_kv.wait()
        cp_q.wait()
        rem = jnp.minimum(nloc - base, CE)

        def edge_body(e, carry2):
            d_loc = dst_l[pl.ds(base + e, DK)][0] - start
            zvec = jnp.zeros((DK,), jnp.float32)
            for h in range(H):
                kvh = kv_buf[e, pl.ds(DK * h, DK)]
                qvh = q_buf[e, pl.ds(DK * h, DK)]
                vvh = kv_buf[e, pl.ds(NDIM + DK * h, DK)]
                p = kvh * qvh
                for sx in shuf:
                    p = p + p.at[sx].get(mode="promise_in_bounds")
                svec = jnp.exp(jnp.clip(p * 0.25, -5.0, 5.0))
                acc[d_loc, pl.ds(DK * h, DK)] = (
                    acc[d_loc, pl.ds(DK * h, DK)] + vvh * svec)
                zvec = jnp.where(lane == h, svec, zvec)
            plsc.addupdate_scatter(z_flat, [d_loc * H + lane], zvec)
            return carry2

        lax.fori_loop(0, rem, edge_body, 0)
        return carry

    nchunk = lax.div(nloc + (CE - 1), CE)
    lax.fori_loop(0, nchunk, chunk_body, 0)

    pltpu.sync_copy(acc, wv_hbm.at[pl.ds(start, SZ)])
    pltpu.sync_copy(z_flat, z_hbm.at[wid])


_edge_call = pl.kernel(
    _edge_body,
    out_type=[
        jax.ShapeDtypeStruct((OUTR, NDIM), jnp.float32),
        jax.ShapeDtypeStruct((NW, ZPAD), jnp.float32),
    ],
    mesh=plsc.VectorSubcoreMesh(core_axis_name="c", subcore_axis_name="s"),
    compiler_params=pltpu.CompilerParams(needs_layout_passes=False),
    scratch_types=[
        pltpu.VMEM((MAXE + DK,), jnp.int32),
        pltpu.VMEM((MAXE + DK,), jnp.int32),
        pltpu.VMEM((SCCH,), jnp.int32),
        pltpu.VMEM((SCCH,), jnp.int32),
        pltpu.VMEM((CE, 2 * NDIM), jnp.float32),
        pltpu.VMEM((CE, NDIM), jnp.float32),
        pltpu.VMEM((SZ, NDIM), jnp.float32),
        pltpu.VMEM((ZPAD,), jnp.float32),
        pltpu.SemaphoreType.DMA,
        pltpu.SemaphoreType.DMA,
    ],
)


def _ln_rows(t, g, b):
    m = jnp.mean(t, axis=-1, keepdims=True)
    v = jnp.mean((t - m) ** 2, axis=-1, keepdims=True)
    return (t - m) * lax.rsqrt(v + 1e-5) * g + b


def _post_body(x_ref, wv_ref, z_ref, e16_ref, wo_ref, bo_ref, w1_ref, b1_ref,
               w2_ref, b2_ref, g1_ref, be1_ref, g2_ref, be2_ref, out_ref):
    wv = wv_ref[...]
    z = z_ref[...]
    zrep = jnp.dot(1.0 / (z + 1e-9), e16_ref[...],
                   preferred_element_type=jnp.float32)
    o = wv * zrep
    t = x_ref[...] + jnp.dot(o, wo_ref[...],
                             preferred_element_type=jnp.float32) + bo_ref[...]
    h1 = _ln_rows(t, g1_ref[...], be1_ref[...])
    ff = jnp.maximum(
        jnp.dot(h1, w1_ref[...], preferred_element_type=jnp.float32)
        + b1_ref[...], 0.0)
    ff = jnp.dot(ff, w2_ref[...], preferred_element_type=jnp.float32) + b2_ref[...]
    out_ref[...] = _ln_rows(h1 + ff, g2_ref[...], be2_ref[...])


_post_call = pl.pallas_call(
    _post_body,
    grid=(N // _BR,),
    in_specs=[
        pl.BlockSpec((_BR, EDIM), lambda i: (i, 0)),
        pl.BlockSpec((_BR, NDIM), lambda i: (i, 0)),
        pl.BlockSpec((_BR, H), lambda i: (i, 0)),
        pl.BlockSpec((H, NDIM), lambda i: (0, 0)),
        pl.BlockSpec((NDIM, EDIM), lambda i: (0, 0)),
        pl.BlockSpec((1, EDIM), lambda i: (0, 0)),
        pl.BlockSpec((EDIM, 4 * EDIM), lambda i: (0, 0)),
        pl.BlockSpec((1, 4 * EDIM), lambda i: (0, 0)),
        pl.BlockSpec((4 * EDIM, EDIM), lambda i: (0, 0)),
        pl.BlockSpec((1, EDIM), lambda i: (0, 0)),
        pl.BlockSpec((1, EDIM), lambda i: (0, 0)),
        pl.BlockSpec((1, EDIM), lambda i: (0, 0)),
        pl.BlockSpec((1, EDIM), lambda i: (0, 0)),
        pl.BlockSpec((1, EDIM), lambda i: (0, 0)),
    ],
    out_specs=pl.BlockSpec((_BR, EDIM), lambda i: (i, 0)),
    out_shape=jax.ShapeDtypeStruct((N, EDIM), jnp.float32),
)


def kernel(x, src_x, dst_x, edge_index, Wq, bq, Wk, bk, Wv, bv, Wo, bo,
           W1, b1, W2, b2, ln1_g, ln1_b, ln2_g, ln2_b):
    w_all = jnp.concatenate([Wk, Wv, Wq], axis=1)
    b_all = jnp.concatenate([bk, bv, bq])[None, :]
    kv_mat, q_mat = _qkv_call(x, w_all, b_all)

    src = edge_index[0].astype(jnp.int32)
    dst = edge_index[1].astype(jnp.int32)
    zerof = jnp.zeros((SZ, NDIM), jnp.float32)
    zeroi = jnp.zeros((MAXE + DK,), jnp.int32)
    wv_full, z_full = _edge_call(kv_mat, q_mat, src, dst, zerof, zeroi)

    wv2 = wv_full[:N]
    z2 = z_full[:, : SZ * H].reshape(OUTR, H)[:N]
    e16 = (jnp.arange(NDIM)[None, :] // DK
           == jnp.arange(H)[:, None]).astype(jnp.float32)

    out = _post_call(x, wv2, z2, e16, Wo, bo[None], W1, b1[None], W2, b2[None],
                     ln1_g[None], ln1_b[None], ln2_g[None], ln2_b[None])
    return (out, src_x, dst_x)


# speedup vs baseline: 15.6200x; 1.1388x over previous
import jax
import jax.numpy as jnp
from jax import lax
from jax.experimental import pallas as pl
from jax.experimental.pallas import tpu as pltpu
from jax.experimental.pallas import tpu_sc as plsc

N = 10000
E = 320000
EDIM = 128
NDIM = 128
H = 8
DK = 16
NC = 2
NS = 16
NW = NC * NS
EPW = E // NW
C = 40
NCHUNK = EPW // C
ROW_W = H + 1


_BR = 1000


def _qkv_body(x_ref, w_ref, b_ref, kv_ref, q_ref):
    y = jnp.dot(x_ref[...], w_ref[...], preferred_element_type=jnp.float32)
    y = y + b_ref[...]
    kv_ref[...] = y[:, : 2 * NDIM]
    q_ref[...] = y[:, 2 * NDIM :]


_qkv_call = pl.pallas_call(
    _qkv_body,
    grid=(N // _BR,),
    in_specs=[
        pl.BlockSpec((_BR, EDIM), lambda i: (i, 0)),
        pl.BlockSpec((EDIM, 3 * NDIM), lambda i: (0, 0)),
        pl.BlockSpec((1, 3 * NDIM), lambda i: (0, 0)),
    ],
    out_specs=[
        pl.BlockSpec((_BR, 2 * NDIM), lambda i: (i, 0)),
        pl.BlockSpec((_BR, NDIM), lambda i: (i, 0)),
    ],
    out_shape=[
        jax.ShapeDtypeStruct((N, 2 * NDIM), jnp.float32),
        jax.ShapeDtypeStruct((N, NDIM), jnp.float32),
    ],
)


SZ = 320
OUTR = NW * SZ
MAXE = 11264
SCCH = 1280
CE = 64
ZPAD = SZ * H
LSZ = MAXE + 2 * CE + DK


def _edge_body(kv_hbm, q_hbm, src_hbm, dst_hbm, zerof_hbm, zeroi_hbm,
               wv_hbm, z_hbm,
               src_l, dst_l, sbuf, dbuf, kv0, kv1, q0, q1, acc, z_flat,
               sem_kv0, sem_kv1, sem_q0, sem_q1):
    c = lax.axis_index("c")
    s = lax.axis_index("s")
    wid = c * NS + s
    start = wid * SZ

    pltpu.sync_copy(zerof_hbm, acc)
    pltpu.sync_copy(zeroi_hbm, src_l)
    pltpu.sync_copy(zeroi_hbm, dst_l)

    lane = lax.iota(jnp.int32, DK)
    lane15 = jnp.full((DK,), DK - 1, jnp.int32)
    zerov = jnp.zeros((DK,), jnp.float32)

    def zloop(t, carry):
        z_flat[pl.ds(t * DK, DK)] = zerov
        return carry

    lax.fori_loop(0, ZPAD // DK, zloop, 0)

    def scan_blk(b, off):
        pltpu.sync_copy(src_hbm.at[pl.ds(b * SCCH, SCCH)], sbuf)
        pltpu.sync_copy(dst_hbm.at[pl.ds(b * SCCH, SCCH)], dbuf)

        def scan16(t, off2):
            dv = dbuf[pl.ds(t * DK, DK)]
            sv = sbuf[pl.ds(t * DK, DK)]
            loc = dv - start
            m = jnp.logical_and(loc >= 0, loc < SZ)
            pos = off2 + plsc.cumsum(m.astype(jnp.int32)) - 1
            plsc.store_scatter(dst_l, [pos], dv, mask=m)
            plsc.store_scatter(src_l, [pos], sv, mask=m)
            return off2 + plsc.all_reduce_population_count(m)

        return lax.fori_loop(0, SCCH // DK, scan16, off)

    off = lax.fori_loop(0, E // SCCH, scan_blk, jnp.zeros((DK,), jnp.int32))
    nloc = off[0]

    def issue(idx, kvb, qb, skv, sq):
        b = idx * CE
        pltpu.async_copy(kv_hbm.at[src_l.at[pl.ds(b, CE)]], kvb, skv)
        pltpu.async_copy(q_hbm.at[dst_l.at[pl.ds(b, CE)]], qb, sq)

    def drain(kvb, qb, skv, sq):
        pltpu.make_async_copy(kv_hbm.at[src_l.at[pl.ds(0, CE)]], kvb, skv).wait()
        pltpu.make_async_copy(q_hbm.at[dst_l.at[pl.ds(0, CE)]], qb, sq).wait()

    def compute(ci, kvb, qb):
        base = ci * CE
        rem = jnp.clip(nloc - base, 0, CE)

        def edge_body(e, carry2):
            d_loc = dst_l[pl.ds(base + e, DK)][0] - start
            zvec = jnp.zeros((DK,), jnp.float32)
            for h in range(H):
                kvh = kvb[e, pl.ds(DK * h, DK)]
                qvh = qb[e, pl.ds(DK * h, DK)]
                vvh = kvb[e, pl.ds(NDIM + DK * h, DK)]
                cs = plsc.cumsum(kvh * qvh)
                sv = jnp.exp(jnp.clip(cs, -5.0, 5.0))
                bs = sv.at[lane15].get(mode="promise_in_bounds")
                acc[d_loc, pl.ds(DK * h, DK)] = (
                    acc[d_loc, pl.ds(DK * h, DK)] + vvh * bs)
                zvec = jnp.where(lane == h, bs, zvec)
            plsc.addupdate_scatter(z_flat, [d_loc * H + lane], zvec)
            return carry2

        lax.fori_loop(0, rem, edge_body, 0)

    nchunk = lax.div(nloc + (CE - 1), CE)
    nbig = lax.div(nchunk + 1, 2)
    issue(0, kv0, q0, sem_kv0, sem_q0)

    def pair_body(j, carry):
        c0 = 2 * j
        issue(c0 + 1, kv1, q1, sem_kv1, sem_q1)
        drain(kv0, q0, sem_kv0, sem_q0)
        compute(c0, kv0, q0)
        issue(c0 + 2, kv0, q0, sem_kv0, sem_q0)
        drain(kv1, q1, sem_kv1, sem_q1)
        compute(c0 + 1, kv1, q1)
        return carry

    lax.fori_loop(0, nbig, pair_body, 0)
    drain(kv0, q0, sem_kv0, sem_q0)

    pltpu.sync_copy(acc, wv_hbm.at[pl.ds(start, SZ)])
    pltpu.sync_copy(z_flat, z_hbm.at[wid])


_edge_call = pl.kernel(
    _edge_body,
    out_type=[
        jax.ShapeDtypeStruct((OUTR, NDIM), jnp.float32),
        jax.ShapeDtypeStruct((NW, ZPAD), jnp.float32),
    ],
    mesh=plsc.VectorSubcoreMesh(core_axis_name="c", subcore_axis_name="s"),
    compiler_params=pltpu.CompilerParams(needs_layout_passes=False),
    scratch_types=[
        pltpu.VMEM((LSZ,), jnp.int32),
        pltpu.VMEM((LSZ,), jnp.int32),
        pltpu.VMEM((SCCH,), jnp.int32),
        pltpu.VMEM((SCCH,), jnp.int32),
        pltpu.VMEM((CE, 2 * NDIM), jnp.float32),
        pltpu.VMEM((CE, 2 * NDIM), jnp.float32),
        pltpu.VMEM((CE, NDIM), jnp.float32),
        pltpu.VMEM((CE, NDIM), jnp.float32),
        pltpu.VMEM((SZ, NDIM), jnp.float32),
        pltpu.VMEM((ZPAD,), jnp.float32),
        pltpu.SemaphoreType.DMA,
        pltpu.SemaphoreType.DMA,
        pltpu.SemaphoreType.DMA,
        pltpu.SemaphoreType.DMA,
    ],
)


def _ln_rows(t, g, b):
    m = jnp.mean(t, axis=-1, keepdims=True)
    v = jnp.mean((t - m) ** 2, axis=-1, keepdims=True)
    return (t - m) * lax.rsqrt(v + 1e-5) * g + b


def _post_body(x_ref, wv_ref, z_ref, e16_ref, wo_ref, bo_ref, w1_ref, b1_ref,
               w2_ref, b2_ref, g1_ref, be1_ref, g2_ref, be2_ref, out_ref):
    wv = wv_ref[...]
    z = z_ref[...]
    zrep = jnp.dot(1.0 / (z + 1e-9), e16_ref[...],
                   preferred_element_type=jnp.float32)
    o = wv * zrep
    t = x_ref[...] + jnp.dot(o, wo_ref[...],
                             preferred_element_type=jnp.float32) + bo_ref[...]
    h1 = _ln_rows(t, g1_ref[...], be1_ref[...])
    ff = jnp.maximum(
        jnp.dot(h1, w1_ref[...], preferred_element_type=jnp.float32)
        + b1_ref[...], 0.0)
    ff = jnp.dot(ff, w2_ref[...], preferred_element_type=jnp.float32) + b2_ref[...]
    out_ref[...] = _ln_rows(h1 + ff, g2_ref[...], be2_ref[...])


_post_call = pl.pallas_call(
    _post_body,
    grid=(N // _BR,),
    in_specs=[
        pl.BlockSpec((_BR, EDIM), lambda i: (i, 0)),
        pl.BlockSpec((_BR, NDIM), lambda i: (i, 0)),
        pl.BlockSpec((_BR, H), lambda i: (i, 0)),
        pl.BlockSpec((H, NDIM), lambda i: (0, 0)),
        pl.BlockSpec((NDIM, EDIM), lambda i: (0, 0)),
        pl.BlockSpec((1, EDIM), lambda i: (0, 0)),
        pl.BlockSpec((EDIM, 4 * EDIM), lambda i: (0, 0)),
        pl.BlockSpec((1, 4 * EDIM), lambda i: (0, 0)),
        pl.BlockSpec((4 * EDIM, EDIM), lambda i: (0, 0)),
        pl.BlockSpec((1, EDIM), lambda i: (0, 0)),
        pl.BlockSpec((1, EDIM), lambda i: (0, 0)),
        pl.BlockSpec((1, EDIM), lambda i: (0, 0)),
        pl.BlockSpec((1, EDIM), lambda i: (0, 0)),
        pl.BlockSpec((1, EDIM), lambda i: (0, 0)),
    ],
    out_specs=pl.BlockSpec((_BR, EDIM), lambda i: (i, 0)),
    out_shape=jax.ShapeDtypeStruct((N, EDIM), jnp.float32),
)


def kernel(x, src_x, dst_x, edge_index, Wq, bq, Wk, bk, Wv, bv, Wo, bo,
           W1, b1, W2, b2, ln1_g, ln1_b, ln2_g, ln2_b):
    w_all = jnp.concatenate([Wk * 0.25, Wv, Wq], axis=1)
    b_all = jnp.concatenate([bk * 0.25, bv, bq])[None, :]
    kv_mat, q_mat = _qkv_call(x, w_all, b_all)

    src = edge_index[0].astype(jnp.int32)
    dst = edge_index[1].astype(jnp.int32)
    zerof = jnp.zeros((SZ, NDIM), jnp.float32)
    zeroi = jnp.zeros((LSZ,), jnp.int32)
    wv_full, z_full = _edge_call(kv_mat, q_mat, src, dst, zerof, zeroi)

    wv2 = wv_full[:N]
    z2 = z_full[:, : SZ * H].reshape(OUTR, H)[:N]
    e16 = (jnp.arange(NDIM)[None, :] // DK
           == jnp.arange(H)[:, None]).astype(jnp.float32)

    out = _post_call(x, wv2, z2, e16, Wo, bo[None], W1, b1[None], W2, b2[None],
                     ln1_g[None], ln1_b[None], ln2_g[None], ln2_b[None])
    return (out, src_x, dst_x)


# speedup vs baseline: 28.5086x; 1.8251x over previous
import jax
import jax.numpy as jnp
from jax import lax
from jax.experimental import pallas as pl
from jax.experimental.pallas import tpu as pltpu
from jax.experimental.pallas import tpu_sc as plsc

N = 10000
E = 320000
EDIM = 128
NDIM = 128
H = 8
DK = 16
NC = 2
NS = 16
NW = NC * NS
EPW = E // NW
C = 40
NCHUNK = EPW // C
ROW_W = H + 1


_BR = 1000


def _qkv_body(x_ref, w_ref, b_ref, kv_ref, q_ref):
    y = jnp.dot(x_ref[...], w_ref[...], preferred_element_type=jnp.float32)
    y = y + b_ref[...]
    kv_ref[...] = y[:, : 2 * NDIM]
    q_ref[...] = y[:, 2 * NDIM :]


_qkv_call = pl.pallas_call(
    _qkv_body,
    grid=(N // _BR,),
    in_specs=[
        pl.BlockSpec((_BR, EDIM), lambda i: (i, 0)),
        pl.BlockSpec((EDIM, 3 * NDIM), lambda i: (0, 0)),
        pl.BlockSpec((1, 3 * NDIM), lambda i: (0, 0)),
    ],
    out_specs=[
        pl.BlockSpec((_BR, 2 * NDIM), lambda i: (i, 0)),
        pl.BlockSpec((_BR, NDIM), lambda i: (i, 0)),
    ],
    out_shape=[
        jax.ShapeDtypeStruct((N, 2 * NDIM), jnp.float32),
        jax.ShapeDtypeStruct((N, NDIM), jnp.float32),
    ],
)


SZ = 320
OUTR = NW * SZ
MAXE = 11264
SCCH = 1280
CE = 64
ZPAD = SZ * H
LSZ = MAXE + 2 * CE + DK


def _edge_body(kv_hbm, q_hbm, src_hbm, dst_hbm, zerof_hbm, zeroi_hbm,
               wv_hbm, z_hbm,
               src_l, dst_l, sbuf, dbuf, kv0, kv1, q0, q1, acc, z_flat,
               sem_kv0, sem_kv1, sem_q0, sem_q1):
    c = lax.axis_index("c")
    s = lax.axis_index("s")
    wid = c * NS + s
    start = wid * SZ

    pltpu.sync_copy(zerof_hbm, acc.at[pl.ds(0, SZ)])
    pltpu.sync_copy(zeroi_hbm, src_l)
    pltpu.sync_copy(zeroi_hbm, dst_l)

    lane = lax.iota(jnp.int32, DK)
    lane15 = jnp.full((DK,), DK - 1, jnp.int32)
    zerov = jnp.zeros((DK,), jnp.float32)

    def zloop(t, carry):
        z_flat[pl.ds(t * DK, DK)] = zerov
        return carry

    lax.fori_loop(0, ZPAD // DK, zloop, 0)

    def scan_blk(b, off):
        pltpu.sync_copy(src_hbm.at[pl.ds(b * SCCH, SCCH)], sbuf)
        pltpu.sync_copy(dst_hbm.at[pl.ds(b * SCCH, SCCH)], dbuf)

        def scan16(t, off2):
            dv = dbuf[pl.ds(t * DK, DK)]
            sv = sbuf[pl.ds(t * DK, DK)]
            loc = dv - start
            m = jnp.logical_and(loc >= 0, loc < SZ)
            pos = off2 + plsc.cumsum(m.astype(jnp.int32)) - 1
            plsc.store_scatter(dst_l, [pos], dv, mask=m)
            plsc.store_scatter(src_l, [pos], sv, mask=m)
            return off2 + plsc.all_reduce_population_count(m)

        return lax.fori_loop(0, SCCH // DK, scan16, off)

    off = lax.fori_loop(0, E // SCCH, scan_blk, jnp.zeros((DK,), jnp.int32))
    nloc = off[0]

    def issue(idx, kvb, qb, skv, sq):
        b = idx * CE
        pltpu.async_copy(kv_hbm.at[src_l.at[pl.ds(b, CE)]], kvb, skv)
        pltpu.async_copy(q_hbm.at[dst_l.at[pl.ds(b, CE)]], qb, sq)

    def drain(kvb, qb, skv, sq):
        pltpu.make_async_copy(kv_hbm.at[src_l.at[pl.ds(0, CE)]], kvb, skv).wait()
        pltpu.make_async_copy(q_hbm.at[dst_l.at[pl.ds(0, CE)]], qb, sq).wait()

    def compute(ci, kvb, qb):
        base = ci * CE

        def edge_body(e, carry2):
            valid = base + e < nloc
            d_loc = jnp.where(valid, dst_l[pl.ds(base + e, DK)][0] - start, SZ)
            zbase = jnp.where(valid, d_loc * H, ZPAD)
            zvec = jnp.zeros((DK,), jnp.float32)
            bss = []
            for h in range(H):
                kvh = kvb[e, pl.ds(DK * h, DK)]
                qvh = qb[e, pl.ds(DK * h, DK)]
                cs = plsc.cumsum(kvh * qvh)
                sv = jnp.exp(jnp.clip(cs, -5.0, 5.0))
                bss.append(sv.at[lane15].get(mode="promise_in_bounds"))
            for h in range(H):
                vvh = kvb[e, pl.ds(NDIM + DK * h, DK)]
                acc[d_loc, pl.ds(DK * h, DK)] = (
                    acc[d_loc, pl.ds(DK * h, DK)] + vvh * bss[h])
                zvec = jnp.where(lane == h, bss[h], zvec)
            plsc.addupdate_scatter(z_flat, [zbase + lane], zvec)
            return carry2

        lax.fori_loop(0, CE, edge_body, 0, unroll=2)

    nchunk = lax.div(nloc + (CE - 1), CE)
    nbig = lax.div(nchunk + 1, 2)
    issue(0, kv0, q0, sem_kv0, sem_q0)

    def pair_body(j, carry):
        c0 = 2 * j
        issue(c0 + 1, kv1, q1, sem_kv1, sem_q1)
        drain(kv0, q0, sem_kv0, sem_q0)
        compute(c0, kv0, q0)
        issue(c0 + 2, kv0, q0, sem_kv0, sem_q0)
        drain(kv1, q1, sem_kv1, sem_q1)
        compute(c0 + 1, kv1, q1)
        return carry

    lax.fori_loop(0, nbig, pair_body, 0)
    drain(kv0, q0, sem_kv0, sem_q0)

    pltpu.sync_copy(acc.at[pl.ds(0, SZ)], wv_hbm.at[pl.ds(start, SZ)])
    pltpu.sync_copy(z_flat.at[pl.ds(0, ZPAD)], z_hbm.at[wid])


_edge_call = pl.kernel(
    _edge_body,
    out_type=[
        jax.ShapeDtypeStruct((OUTR, NDIM), jnp.float32),
        jax.ShapeDtypeStruct((NW, ZPAD), jnp.float32),
    ],
    mesh=plsc.VectorSubcoreMesh(core_axis_name="c", subcore_axis_name="s"),
    compiler_params=pltpu.CompilerParams(needs_layout_passes=False),
    scratch_types=[
        pltpu.VMEM((LSZ,), jnp.int32),
        pltpu.VMEM((LSZ,), jnp.int32),
        pltpu.VMEM((SCCH,), jnp.int32),
        pltpu.VMEM((SCCH,), jnp.int32),
        pltpu.VMEM((CE, 2 * NDIM), jnp.float32),
        pltpu.VMEM((CE, 2 * NDIM), jnp.float32),
        pltpu.VMEM((CE, NDIM), jnp.float32),
        pltpu.VMEM((CE, NDIM), jnp.float32),
        pltpu.VMEM((SZ + 8, NDIM), jnp.float32),
        pltpu.VMEM((ZPAD + DK,), jnp.float32),
        pltpu.SemaphoreType.DMA,
        pltpu.SemaphoreType.DMA,
        pltpu.SemaphoreType.DMA,
        pltpu.SemaphoreType.DMA,
    ],
)


def _ln_rows(t, g, b):
    m = jnp.mean(t, axis=-1, keepdims=True)
    v = jnp.mean((t - m) ** 2, axis=-1, keepdims=True)
    return (t - m) * lax.rsqrt(v + 1e-5) * g + b


def _post_body(x_ref, wv_ref, z_ref, e16_ref, wo_ref, bo_ref, w1_ref, b1_ref,
               w2_ref, b2_ref, g1_ref, be1_ref, g2_ref, be2_ref, out_ref):
    wv = wv_ref[...]
    z = z_ref[...]
    zrep = jnp.dot(1.0 / (z + 1e-9), e16_ref[...],
                   preferred_element_type=jnp.float32)
    o = wv * zrep
    t = x_ref[...] + jnp.dot(o, wo_ref[...],
                             preferred_element_type=jnp.float32) + bo_ref[...]
    h1 = _ln_rows(t, g1_ref[...], be1_ref[...])
    ff = jnp.maximum(
        jnp.dot(h1, w1_ref[...], preferred_element_type=jnp.float32)
        + b1_ref[...], 0.0)
    ff = jnp.dot(ff, w2_ref[...], preferred_element_type=jnp.float32) + b2_ref[...]
    out_ref[...] = _ln_rows(h1 + ff, g2_ref[...], be2_ref[...])


_post_call = pl.pallas_call(
    _post_body,
    grid=(N // _BR,),
    in_specs=[
        pl.BlockSpec((_BR, EDIM), lambda i: (i, 0)),
        pl.BlockSpec((_BR, NDIM), lambda i: (i, 0)),
        pl.BlockSpec((_BR, H), lambda i: (i, 0)),
        pl.BlockSpec((H, NDIM), lambda i: (0, 0)),
        pl.BlockSpec((NDIM, EDIM), lambda i: (0, 0)),
        pl.BlockSpec((1, EDIM), lambda i: (0, 0)),
        pl.BlockSpec((EDIM, 4 * EDIM), lambda i: (0, 0)),
        pl.BlockSpec((1, 4 * EDIM), lambda i: (0, 0)),
        pl.BlockSpec((4 * EDIM, EDIM), lambda i: (0, 0)),
        pl.BlockSpec((1, EDIM), lambda i: (0, 0)),
        pl.BlockSpec((1, EDIM), lambda i: (0, 0)),
        pl.BlockSpec((1, EDIM), lambda i: (0, 0)),
        pl.BlockSpec((1, EDIM), lambda i: (0, 0)),
        pl.BlockSpec((1, EDIM), lambda i: (0, 0)),
    ],
    out_specs=pl.BlockSpec((_BR, EDIM), lambda i: (i, 0)),
    out_shape=jax.ShapeDtypeStruct((N, EDIM), jnp.float32),
)


def kernel(x, src_x, dst_x, edge_index, Wq, bq, Wk, bk, Wv, bv, Wo, bo,
           W1, b1, W2, b2, ln1_g, ln1_b, ln2_g, ln2_b):
    w_all = jnp.concatenate([Wk * 0.25, Wv, Wq], axis=1)
    b_all = jnp.concatenate([bk * 0.25, bv, bq])[None, :]
    kv_mat, q_mat = _qkv_call(x, w_all, b_all)

    src = edge_index[0].astype(jnp.int32)
    dst = edge_index[1].astype(jnp.int32)
    zerof = jnp.zeros((SZ, NDIM), jnp.float32)
    zeroi = jnp.zeros((LSZ,), jnp.int32)
    wv_full, z_full = _edge_call(kv_mat, q_mat, src, dst, zerof, zeroi)

    wv2 = wv_full[:N]
    z2 = z_full[:, : SZ * H].reshape(OUTR, H)[:N]
    e16 = (jnp.arange(NDIM)[None, :] // DK
           == jnp.arange(H)[:, None]).astype(jnp.float32)

    out = _post_call(x, wv2, z2, e16, Wo, bo[None], W1, b1[None], W2, b2[None],
                     ln1_g[None], ln1_b[None], ln2_g[None], ln2_b[None])
    return (out, src_x, dst_x)


# speedup vs baseline: 28.5422x; 1.0012x over previous
import jax
import jax.numpy as jnp
from jax import lax
from jax.experimental import pallas as pl
from jax.experimental.pallas import tpu as pltpu
from jax.experimental.pallas import tpu_sc as plsc

N = 10000
E = 320000
EDIM = 128
NDIM = 128
H = 8
DK = 16
NC = 2
NS = 16
NW = NC * NS
EPW = E // NW
C = 40
NCHUNK = EPW // C
ROW_W = H + 1


_BR = 1000


def _qkv_body(x_ref, w_ref, b_ref, kv_ref, q_ref):
    y = jnp.dot(x_ref[...], w_ref[...], preferred_element_type=jnp.float32)
    y = y + b_ref[...]
    kv_ref[...] = y[:, : 2 * NDIM]
    q_ref[...] = y[:, 2 * NDIM :]


_qkv_call = pl.pallas_call(
    _qkv_body,
    grid=(N // _BR,),
    in_specs=[
        pl.BlockSpec((_BR, EDIM), lambda i: (i, 0)),
        pl.BlockSpec((EDIM, 3 * NDIM), lambda i: (0, 0)),
        pl.BlockSpec((1, 3 * NDIM), lambda i: (0, 0)),
    ],
    out_specs=[
        pl.BlockSpec((_BR, 2 * NDIM), lambda i: (i, 0)),
        pl.BlockSpec((_BR, NDIM), lambda i: (i, 0)),
    ],
    out_shape=[
        jax.ShapeDtypeStruct((N, 2 * NDIM), jnp.float32),
        jax.ShapeDtypeStruct((N, NDIM), jnp.float32),
    ],
)


SZ = 320
OUTR = NW * SZ
MAXE = 11264
SCCH = 1280
CE = 64
ZPAD = SZ * H
LSZ = MAXE + 2 * CE + DK


def _edge_body(kv_hbm, q_hbm, src_hbm, dst_hbm, zerof_hbm, zeroi_hbm,
               wv_hbm, z_hbm,
               src_l, dst_l, sbuf, dbuf, kv0, kv1, q0, q1, acc, z_flat,
               sem_kv0, sem_kv1, sem_q0, sem_q1):
    c = lax.axis_index("c")
    s = lax.axis_index("s")
    wid = c * NS + s
    start = wid * SZ

    pltpu.sync_copy(zerof_hbm, acc.at[pl.ds(0, SZ)])
    pltpu.sync_copy(zeroi_hbm, src_l)
    pltpu.sync_copy(zeroi_hbm, dst_l)

    lane = lax.iota(jnp.int32, DK)
    lane15 = jnp.full((DK,), DK - 1, jnp.int32)
    zerov = jnp.zeros((DK,), jnp.float32)

    def zloop(t, carry):
        z_flat[pl.ds(t * DK, DK)] = zerov
        return carry

    lax.fori_loop(0, ZPAD // DK, zloop, 0)

    def scan_blk(b, off):
        pltpu.sync_copy(src_hbm.at[pl.ds(b * SCCH, SCCH)], sbuf)
        pltpu.sync_copy(dst_hbm.at[pl.ds(b * SCCH, SCCH)], dbuf)

        def scan16(t, off2):
            dv = dbuf[pl.ds(t * DK, DK)]
            sv = sbuf[pl.ds(t * DK, DK)]
            loc = dv - start
            m = jnp.logical_and(loc >= 0, loc < SZ)
            pos = off2 + plsc.cumsum(m.astype(jnp.int32)) - 1
            plsc.store_scatter(dst_l, [pos], dv, mask=m)
            plsc.store_scatter(src_l, [pos], sv, mask=m)
            return off2 + plsc.all_reduce_population_count(m)

        return lax.fori_loop(0, SCCH // DK, scan16, off, unroll=2)

    off = lax.fori_loop(0, E // SCCH, scan_blk, jnp.zeros((DK,), jnp.int32))
    nloc = off[0]

    def issue(idx, kvb, qb, skv, sq):
        b = idx * CE
        pltpu.async_copy(kv_hbm.at[src_l.at[pl.ds(b, CE)]], kvb, skv)
        pltpu.async_copy(q_hbm.at[dst_l.at[pl.ds(b, CE)]], qb, sq)

    def drain(kvb, qb, skv, sq):
        pltpu.make_async_copy(kv_hbm.at[src_l.at[pl.ds(0, CE)]], kvb, skv).wait()
        pltpu.make_async_copy(q_hbm.at[dst_l.at[pl.ds(0, CE)]], qb, sq).wait()

    def compute(ci, kvb, qb):
        base = ci * CE

        def edge_body(e, carry2):
            valid = base + e < nloc
            d_loc = jnp.where(valid, dst_l[pl.ds(base + e, DK)][0] - start, SZ)
            zbase = jnp.where(valid, d_loc * H, ZPAD)
            zvec = jnp.zeros((DK,), jnp.float32)
            bss = []
            for h in range(H):
                kvh = kvb[e, pl.ds(DK * h, DK)]
                qvh = qb[e, pl.ds(DK * h, DK)]
                cs = plsc.cumsum(kvh * qvh)
                sv = jnp.exp(jnp.clip(cs, -5.0, 5.0))
                bss.append(sv.at[lane15].get(mode="promise_in_bounds"))
            for h in range(H):
                vvh = kvb[e, pl.ds(NDIM + DK * h, DK)]
                acc[d_loc, pl.ds(DK * h, DK)] = (
                    acc[d_loc, pl.ds(DK * h, DK)] + vvh * bss[h])
                zvec = jnp.where(lane == h, bss[h], zvec)
            plsc.addupdate_scatter(z_flat, [zbase + lane], zvec)
            return carry2

        lax.fori_loop(0, CE, edge_body, 0, unroll=4)

    nchunk = lax.div(nloc + (CE - 1), CE)
    nbig = lax.div(nchunk + 1, 2)
    issue(0, kv0, q0, sem_kv0, sem_q0)

    def pair_body(j, carry):
        c0 = 2 * j
        issue(c0 + 1, kv1, q1, sem_kv1, sem_q1)
        drain(kv0, q0, sem_kv0, sem_q0)
        compute(c0, kv0, q0)
        issue(c0 + 2, kv0, q0, sem_kv0, sem_q0)
        drain(kv1, q1, sem_kv1, sem_q1)
        compute(c0 + 1, kv1, q1)
        return carry

    lax.fori_loop(0, nbig, pair_body, 0)
    drain(kv0, q0, sem_kv0, sem_q0)

    pltpu.sync_copy(acc.at[pl.ds(0, SZ)], wv_hbm.at[pl.ds(start, SZ)])
    pltpu.sync_copy(z_flat.at[pl.ds(0, ZPAD)], z_hbm.at[wid])


_edge_call = pl.kernel(
    _edge_body,
    out_type=[
        jax.ShapeDtypeStruct((OUTR, NDIM), jnp.float32),
        jax.ShapeDtypeStruct((NW, ZPAD), jnp.float32),
    ],
    mesh=plsc.VectorSubcoreMesh(core_axis_name="c", subcore_axis_name="s"),
    compiler_params=pltpu.CompilerParams(needs_layout_passes=False),
    scratch_types=[
        pltpu.VMEM((LSZ,), jnp.int32),
        pltpu.VMEM((LSZ,), jnp.int32),
        pltpu.VMEM((SCCH,), jnp.int32),
        pltpu.VMEM((SCCH,), jnp.int32),
        pltpu.VMEM((CE, 2 * NDIM), jnp.float32),
        pltpu.VMEM((CE, 2 * NDIM), jnp.float32),
        pltpu.VMEM((CE, NDIM), jnp.float32),
        pltpu.VMEM((CE, NDIM), jnp.float32),
        pltpu.VMEM((SZ + 8, NDIM), jnp.float32),
        pltpu.VMEM((ZPAD + DK,), jnp.float32),
        pltpu.SemaphoreType.DMA,
        pltpu.SemaphoreType.DMA,
        pltpu.SemaphoreType.DMA,
        pltpu.SemaphoreType.DMA,
    ],
)


def _ln_rows(t, g, b):
    m = jnp.mean(t, axis=-1, keepdims=True)
    v = jnp.mean((t - m) ** 2, axis=-1, keepdims=True)
    return (t - m) * lax.rsqrt(v + 1e-5) * g + b


def _post_body(x_ref, wv_ref, z_ref, e16_ref, wo_ref, bo_ref, w1_ref, b1_ref,
               w2_ref, b2_ref, g1_ref, be1_ref, g2_ref, be2_ref, out_ref):
    wv = wv_ref[...]
    z = z_ref[...]
    zrep = jnp.dot(1.0 / (z + 1e-9), e16_ref[...],
                   preferred_element_type=jnp.float32)
    o = wv * zrep
    t = x_ref[...] + jnp.dot(o, wo_ref[...],
                             preferred_element_type=jnp.float32) + bo_ref[...]
    h1 = _ln_rows(t, g1_ref[...], be1_ref[...])
    ff = jnp.maximum(
        jnp.dot(h1, w1_ref[...], preferred_element_type=jnp.float32)
        + b1_ref[...], 0.0)
    ff = jnp.dot(ff, w2_ref[...], preferred_element_type=jnp.float32) + b2_ref[...]
    out_ref[...] = _ln_rows(h1 + ff, g2_ref[...], be2_ref[...])


_post_call = pl.pallas_call(
    _post_body,
    grid=(N // _BR,),
    in_specs=[
        pl.BlockSpec((_BR, EDIM), lambda i: (i, 0)),
        pl.BlockSpec((_BR, NDIM), lambda i: (i, 0)),
        pl.BlockSpec((_BR, H), lambda i: (i, 0)),
        pl.BlockSpec((H, NDIM), lambda i: (0, 0)),
        pl.BlockSpec((NDIM, EDIM), lambda i: (0, 0)),
        pl.BlockSpec((1, EDIM), lambda i: (0, 0)),
        pl.BlockSpec((EDIM, 4 * EDIM), lambda i: (0, 0)),
        pl.BlockSpec((1, 4 * EDIM), lambda i: (0, 0)),
        pl.BlockSpec((4 * EDIM, EDIM), lambda i: (0, 0)),
        pl.BlockSpec((1, EDIM), lambda i: (0, 0)),
        pl.BlockSpec((1, EDIM), lambda i: (0, 0)),
        pl.BlockSpec((1, EDIM), lambda i: (0, 0)),
        pl.BlockSpec((1, EDIM), lambda i: (0, 0)),
        pl.BlockSpec((1, EDIM), lambda i: (0, 0)),
    ],
    out_specs=pl.BlockSpec((_BR, EDIM), lambda i: (i, 0)),
    out_shape=jax.ShapeDtypeStruct((N, EDIM), jnp.float32),
)


def kernel(x, src_x, dst_x, edge_index, Wq, bq, Wk, bk, Wv, bv, Wo, bo,
           W1, b1, W2, b2, ln1_g, ln1_b, ln2_g, ln2_b):
    w_all = jnp.concatenate([Wk * 0.25, Wv, Wq], axis=1)
    b_all = jnp.concatenate([bk * 0.25, bv, bq])[None, :]
    kv_mat, q_mat = _qkv_call(x, w_all, b_all)

    src = edge_index[0].astype(jnp.int32)
    dst = edge_index[1].astype(jnp.int32)
    zerof = jnp.zeros((SZ, NDIM), jnp.float32)
    zeroi = jnp.zeros((LSZ,), jnp.int32)
    wv_full, z_full = _edge_call(kv_mat, q_mat, src, dst, zerof, zeroi)

    wv2 = wv_full[:N]
    z2 = z_full[:, : SZ * H].reshape(OUTR, H)[:N]
    e16 = (jnp.arange(NDIM)[None, :] // DK
           == jnp.arange(H)[:, None]).astype(jnp.float32)

    out = _post_call(x, wv2, z2, e16, Wo, bo[None], W1, b1[None], W2, b2[None],
                     ln1_g[None], ln1_b[None], ln2_g[None], ln2_b[None])
    return (out, src_x, dst_x)
